# Initial kernel scaffold; baseline (speedup 1.0000x reference)
import jax
import jax.numpy as jnp
from jax.experimental import pallas as pl


def kernel(t, beta_schedule):
    raise NotImplementedError("write your pallas kernel here")


# speedup vs baseline: 4.5614x; 4.5614x over previous
import functools

import jax
import jax.numpy as jnp
from jax import lax
from jax.experimental import pallas as pl
from jax.experimental.pallas import tpu as pltpu
from jax.experimental.pallas import tpu_sc as plsc

_N_TABLE = 1000
_B = 16384
_NC = 2
_NS = 16
_NW = _NC * _NS
_L = 16
_B_PER_W = _B // _NW


def _gather_body(t_hbm, table_hbm, out_hbm, idx_v, vals_v, tab_v):
    wid = lax.axis_index("s") * _NC + lax.axis_index("c")
    base = wid * _B_PER_W
    pltpu.sync_copy(table_hbm, tab_v)
    pltpu.sync_copy(t_hbm.at[pl.ds(base, _B_PER_W)], idx_v)

    def step(i, carry):
        idx = idx_v[pl.ds(i * _L, _L)]
        vals_v[pl.ds(i * _L, _L)] = plsc.load_gather(tab_v, [idx])
        return carry

    lax.fori_loop(0, _B_PER_W // _L, step, 0, unroll=8)
    pltpu.sync_copy(vals_v, out_hbm.at[pl.ds(base, _B_PER_W)])


_gather = pl.kernel(
    _gather_body,
    out_type=jax.ShapeDtypeStruct((_B,), jnp.float32),
    mesh=plsc.VectorSubcoreMesh(core_axis_name="c", subcore_axis_name="s"),
    scratch_types=[
        pltpu.VMEM((_B_PER_W,), jnp.int32),
        pltpu.VMEM((_B_PER_W,), jnp.float32),
        pltpu.VMEM((_N_TABLE,), jnp.float32),
    ],
    compiler_params=pltpu.CompilerParams(needs_layout_passes=False),
)


@jax.jit
def kernel(t, beta_schedule):
    return _gather(t.astype(jnp.int32), beta_schedule)


# speedup vs baseline: 4.6702x; 1.0239x over previous
import functools

import jax
import jax.numpy as jnp
from jax import lax
from jax.experimental import pallas as pl
from jax.experimental.pallas import tpu as pltpu
from jax.experimental.pallas import tpu_sc as plsc

_N_TABLE = 1000
_B = 16384
_NC = 2
_NS = 16
_NW = _NC * _NS
_L = 16
_B_PER_W = _B // _NW


def _gather_body(t_hbm, table_hbm, out_hbm, idx_v, vals_v, tab_v, sem_t, sem_i):
    wid = lax.axis_index("s") * _NC + lax.axis_index("c")
    base = wid * _B_PER_W
    cp_tab = pltpu.async_copy(table_hbm, tab_v, sem_t)
    cp_idx = pltpu.async_copy(t_hbm.at[pl.ds(base, _B_PER_W)], idx_v, sem_i)
    cp_tab.wait()
    cp_idx.wait()

    def step(i, carry):
        idx = idx_v[pl.ds(i * _L, _L)]
        vals_v[pl.ds(i * _L, _L)] = plsc.load_gather(tab_v, [idx])
        return carry

    lax.fori_loop(0, _B_PER_W // _L, step, 0, unroll=16)
    pltpu.sync_copy(vals_v, out_hbm.at[pl.ds(base, _B_PER_W)])


_gather = pl.kernel(
    _gather_body,
    out_type=jax.ShapeDtypeStruct((_B,), jnp.float32),
    mesh=plsc.VectorSubcoreMesh(core_axis_name="c", subcore_axis_name="s"),
    scratch_types=[
        pltpu.VMEM((_B_PER_W,), jnp.int32),
        pltpu.VMEM((_B_PER_W,), jnp.float32),
        pltpu.VMEM((_N_TABLE,), jnp.float32),
        pltpu.SemaphoreType.DMA,
        pltpu.SemaphoreType.DMA,
    ],
    compiler_params=pltpu.CompilerParams(needs_layout_passes=False),
)


@jax.jit
def kernel(t, beta_schedule):
    return _gather(t.astype(jnp.int32), beta_schedule)


# speedup vs baseline: 5.0462x; 1.0805x over previous
import functools

import jax
import jax.numpy as jnp
from jax import lax
from jax.experimental import pallas as pl
from jax.experimental.pallas import tpu as pltpu
from jax.experimental.pallas import tpu_sc as plsc

_N_TABLE = 1000
_B = 16384
_NC = 1
_NS = 16
_NW = _NC * _NS
_L = 16
_B_PER_W = _B // _NW


def _gather_body(t_hbm, table_hbm, out_hbm, idx_v, vals_v, tab_v, sem_t, sem_i):
    wid = lax.axis_index("s") * _NC + lax.axis_index("c")
    base = wid * _B_PER_W
    cp_tab = pltpu.async_copy(table_hbm, tab_v, sem_t)
    cp_idx = pltpu.async_copy(t_hbm.at[pl.ds(base, _B_PER_W)], idx_v, sem_i)
    cp_tab.wait()
    cp_idx.wait()

    def step(i, carry):
        idx = idx_v[pl.ds(i * _L, _L)]
        vals_v[pl.ds(i * _L, _L)] = plsc.load_gather(tab_v, [idx])
        return carry

    lax.fori_loop(0, _B_PER_W // _L, step, 0, unroll=16)
    pltpu.sync_copy(vals_v, out_hbm.at[pl.ds(base, _B_PER_W)])


_gather = pl.kernel(
    _gather_body,
    out_type=jax.ShapeDtypeStruct((_B,), jnp.float32),
    mesh=plsc.VectorSubcoreMesh(
        core_axis_name="c", subcore_axis_name="s", num_cores=_NC
    ),
    scratch_types=[
        pltpu.VMEM((_B_PER_W,), jnp.int32),
        pltpu.VMEM((_B_PER_W,), jnp.float32),
        pltpu.VMEM((_N_TABLE,), jnp.float32),
        pltpu.SemaphoreType.DMA,
        pltpu.SemaphoreType.DMA,
    ],
    compiler_params=pltpu.CompilerParams(needs_layout_passes=False),
)


@jax.jit
def kernel(t, beta_schedule):
    return _gather(t.astype(jnp.int32), beta_schedule)


# speedup vs baseline: 5.0645x; 1.0036x over previous
import functools

import jax
import jax.numpy as jnp
from jax import lax
from jax.experimental import pallas as pl
from jax.experimental.pallas import tpu as pltpu
from jax.experimental.pallas import tpu_sc as plsc

_N_TABLE = 1000
_B = 16384
_NC = 1
_NS = 16
_NW = _NC * _NS
_L = 16
_B_PER_W = _B // _NW


def _gather_body(t_hbm, table_hbm, out_hbm, idx_v, vals_v, tab_v, sem_t, sem_i):
    wid = lax.axis_index("s") * _NC + lax.axis_index("c")
    base = wid * _B_PER_W
    cp_tab = pltpu.async_copy(table_hbm, tab_v, sem_t)
    cp_idx = pltpu.async_copy(t_hbm.at[pl.ds(base, _B_PER_W)], idx_v, sem_i)
    cp_tab.wait()
    cp_idx.wait()

    def step(i, carry):
        idx = idx_v[pl.ds(i * _L, _L)]
        vals_v[pl.ds(i * _L, _L)] = plsc.load_gather(tab_v, [idx])
        return carry

    lax.fori_loop(0, _B_PER_W // _L, step, 0, unroll=4)
    pltpu.sync_copy(vals_v, out_hbm.at[pl.ds(base, _B_PER_W)])


_gather = pl.kernel(
    _gather_body,
    out_type=jax.ShapeDtypeStruct((_B,), jnp.float32),
    mesh=plsc.VectorSubcoreMesh(
        core_axis_name="c", subcore_axis_name="s", num_cores=_NC
    ),
    scratch_types=[
        pltpu.VMEM((_B_PER_W,), jnp.int32),
        pltpu.VMEM((_B_PER_W,), jnp.float32),
        pltpu.VMEM((_N_TABLE,), jnp.float32),
        pltpu.SemaphoreType.DMA,
        pltpu.SemaphoreType.DMA,
    ],
    compiler_params=pltpu.CompilerParams(needs_layout_passes=False),
)


@jax.jit
def kernel(t, beta_schedule):
    return _gather(t.astype(jnp.int32), beta_schedule)
